# baseline (speedup 1.0000x reference)
import functools

import jax
import jax.numpy as jnp
from jax import lax
from jax.experimental import pallas as pl
from jax.experimental.pallas import tpu as pltpu
from jax.experimental.pallas import tpu_sc as plsc

_DROP_PROB = 0.1
_MIN_KEEP = 1


def _select_kernel(imp_ref, noise_ref, keep_ref, scaled_ref, *, k, scale):
    rows, lanes = imp_ref.shape
    n = rows * lanes
    score = 1.0 / (imp_ref[...] + 1e-8) + noise_ref[...]
    bits = jax.lax.bitcast_convert_type(score, jnp.int32)

    def _bits_body(_, carry):
        lo, hi = carry
        mid = lo + (hi - lo) // 2
        cnt = jnp.sum((bits <= mid).astype(jnp.int32))
        ge = cnt >= k
        return jnp.where(ge, lo, mid + 1), jnp.where(ge, mid, hi)

    t, _ = jax.lax.fori_loop(
        0, 31, _bits_body, (jnp.int32(0), jnp.int32(2**31 - 1))
    )

    n_less = jnp.sum((bits < t).astype(jnp.int32))
    rem = k - n_less
    eq = bits == t
    idx = (
        jax.lax.broadcasted_iota(jnp.int32, (rows, lanes), 0) * lanes
        + jax.lax.broadcasted_iota(jnp.int32, (rows, lanes), 1)
    )

    def _idx_body(_, carry):
        lo, hi = carry
        mid = lo + (hi - lo) // 2
        cnt = jnp.sum((eq & (idx < mid)).astype(jnp.int32))
        ge = cnt >= rem
        return jnp.where(ge, lo, mid + 1), jnp.where(ge, mid, hi)

    m, _ = jax.lax.fori_loop(0, 16, _idx_body, (jnp.int32(0), jnp.int32(n)))

    keep = (bits < t) | (eq & (idx < m))
    keep_f = keep.astype(keep_ref.dtype)
    keep_ref[...] = keep_f
    scaled_ref[...] = keep_f * scale


def _sc_select_body(imp_hbm, noise_hbm, keep_hbm, scaled_hbm,
                    imp_v, noise_v, bits_v, hist_v, hist256_v, ghist_v,
                    keep_v, scaled_v, row_v, shist, seq, alleq_v, eqv_v,
                    *, n, k, scale, tiles):
    sl = n // tiles
    nv = sl // 16
    cid = lax.axis_index("c")
    wid = lax.axis_index("s")
    base_off = wid * sl
    lane = lax.iota(jnp.int32, 16)
    ones_i = jnp.ones((16,), jnp.int32)

    pltpu.sync_copy(imp_hbm.at[pl.ds(base_off, sl)], imp_v)
    pltpu.sync_copy(noise_hbm.at[pl.ds(base_off, sl)], noise_v)

    def _compute_bits(j, _):
        v = imp_v[pl.ds(j * 16, 16)]
        s = 1.0 / (v + 1e-8) + noise_v[pl.ds(j * 16, 16)]
        bits_v[pl.ds(j * 16, 16)] = lax.bitcast_convert_type(s, jnp.int32)
        return 0

    lax.fori_loop(0, nv, _compute_bits, 0)

    prefix = jnp.int32(0)
    base = jnp.int32(0)
    for r in range(4):
        sh = 24 - 8 * r

        def _zero_hist(j, _):
            hist_v[pl.ds(j * 16, 16)] = jnp.zeros((16,), jnp.int32)
            return 0

        lax.fori_loop(0, (16 * 256) // 16, _zero_hist, 0)

        def _hist(j, _, _sh=sh, _r=r, _prefix=prefix):
            v = bits_v[pl.ds(j * 16, 16)]
            if _r == 0:
                active = v == v
            else:
                active = lax.shift_right_logical(v, _sh + 8) == _prefix
            bin_ = jnp.bitwise_and(lax.shift_right_logical(v, _sh), 255)
            plsc.addupdate_scatter(hist_v, [lane * 256 + bin_], ones_i,
                                   mask=active)
            return 0

        lax.fori_loop(0, nv, _hist, 0)

        def _reduce_lanes(cc, _):
            acc = jnp.zeros((16,), jnp.int32)
            for l in range(16):
                acc = acc + hist_v[pl.ds(l * 256 + cc * 16, 16)]
            hist256_v[pl.ds(cc * 16, 16)] = acc
            return 0

        lax.fori_loop(0, 16, _reduce_lanes, 0)

        pltpu.sync_copy(hist256_v, shist.at[wid])
        plsc.subcore_barrier()
        pltpu.sync_copy(shist, row_v)

        def _reduce_tiles(cc, _):
            acc = jnp.zeros((16,), jnp.int32)
            for t in range(tiles):
                acc = acc + row_v[t, pl.ds(cc * 16, 16)]
            ghist_v[pl.ds(cc * 16, 16)] = acc
            return 0

        lax.fori_loop(0, 16, _reduce_tiles, 0)

        def _scan(cc, carry):
            cum, found, bstar, below = carry
            chunk = ghist_v[pl.ds(cc * 16, 16)]
            csum = plsc.cumsum(chunk)
            tot = jnp.sum(chunk)
            meets = (base + cum + csum) >= k
            p = jnp.sum(jnp.where(meets, 0, 1))
            before = jnp.sum(jnp.where(meets, 0, chunk))
            has = (base + cum + tot) >= k
            is_new = jnp.logical_and(has, jnp.logical_not(found))
            bstar = jnp.where(is_new, cc * 16 + p, bstar)
            below = jnp.where(is_new, cum + before, below)
            return cum + tot, jnp.logical_or(found, has), bstar, below

        _, _, bstar, below = lax.fori_loop(
            0, 16, _scan,
            (jnp.int32(0), jnp.bool_(False), jnp.int32(0), jnp.int32(0)),
        )
        prefix = prefix * 256 + bstar
        base = base + below

    t_bits = prefix
    rem = k - base

    def _eq_count(j, acc):
        v = bits_v[pl.ds(j * 16, 16)]
        return acc + jnp.sum(jnp.where(v == t_bits, 1, 0))

    local_eq = lax.fori_loop(0, nv, _eq_count, jnp.int32(0))
    eqv_v[...] = jnp.broadcast_to(local_eq, (16,))
    pltpu.sync_copy(eqv_v, seq.at[wid])
    plsc.subcore_barrier()
    pltpu.sync_copy(seq, alleq_v)

    eq_before = jnp.int32(0)
    for t in range(tiles):
        cnt_t = jnp.max(alleq_v[t, :])
        eq_before = eq_before + jnp.where(jnp.int32(t) < wid, cnt_t, 0)

    quota = jnp.maximum(jnp.minimum(rem - eq_before, local_eq), 0)

    def _emit(j, carry):
        v = bits_v[pl.ds(j * 16, 16)]
        eq = v == t_bits
        less = v < t_bits
        eqi = jnp.where(eq, 1, 0)
        csum = plsc.cumsum(eqi)
        keep_extra = jnp.logical_and(eq, (carry + csum) <= quota)
        keep_f = jnp.where(jnp.logical_or(less, keep_extra),
                           jnp.float32(1.0), jnp.float32(0.0))
        keep_v[pl.ds(j * 16, 16)] = keep_f
        scaled_v[pl.ds(j * 16, 16)] = keep_f * scale
        return carry + jnp.sum(eqi)

    lax.fori_loop(0, nv, _emit, jnp.int32(0))

    @pl.when(cid == 0)
    def _():
        pltpu.sync_copy(keep_v, keep_hbm.at[pl.ds(base_off, sl)])
        pltpu.sync_copy(scaled_v, scaled_hbm.at[pl.ds(base_off, sl)])


def _sc_select(importance, noise, *, n, k, scale):
    tiles = 16
    sl = n // tiles
    mesh = plsc.VectorSubcoreMesh(core_axis_name="c", subcore_axis_name="s")
    f32, i32 = jnp.float32, jnp.int32
    body = functools.partial(_sc_select_body, n=n, k=k, scale=scale,
                             tiles=tiles)
    return pl.kernel(
        body,
        mesh=mesh,
        compiler_params=pltpu.CompilerParams(needs_layout_passes=False),
        out_type=(
            jax.ShapeDtypeStruct((n,), f32),
            jax.ShapeDtypeStruct((n,), f32),
        ),
        scratch_types=[
            pltpu.VMEM((sl,), f32),
            pltpu.VMEM((sl,), f32),
            pltpu.VMEM((sl,), i32),
            pltpu.VMEM((16 * 256,), i32),
            pltpu.VMEM((256,), i32),
            pltpu.VMEM((256,), i32),
            pltpu.VMEM((sl,), f32),
            pltpu.VMEM((sl,), f32),
            pltpu.VMEM((tiles, 256), i32),
            pltpu.VMEM_SHARED((tiles, 256), i32),
            pltpu.VMEM_SHARED((tiles, 16), i32),
            pltpu.VMEM((tiles, 16), i32),
            pltpu.VMEM((16,), i32),
        ],
    )(importance, noise)


def _apply_kernel(x_ref, m_ref, o_ref):
    o_ref[...] = x_ref[...] * m_ref[0:1, :][:, None, :]


def kernel(x, importance):
    b, n, c = x.shape
    num_keep = max(_MIN_KEEP, int(n * (1.0 - _DROP_PROB)))
    scale = n / num_keep
    noise = (
        jax.random.uniform(jax.random.key(42), importance.shape,
                           dtype=importance.dtype)
        * 0.5
    )
    keep_mask, scaled = _sc_select(importance, noise, n=n, k=num_keep,
                                   scale=scale)
    xt = jnp.transpose(x, (0, 2, 1))
    mask_row = jnp.broadcast_to(scaled[None, :], (8, n))

    w = n
    out_t = pl.pallas_call(
        _apply_kernel,
        grid=(n // w, b // 4),
        in_specs=[
            pl.BlockSpec((4, c, w), lambda j, i: (i, 0, j)),
            pl.BlockSpec((8, w), lambda j, i: (0, j)),
        ],
        out_specs=pl.BlockSpec((4, c, w), lambda j, i: (i, 0, j)),
        out_shape=jax.ShapeDtypeStruct((b, c, n), x.dtype),
        compiler_params=pltpu.CompilerParams(
            dimension_semantics=("arbitrary", "arbitrary"),
        ),
    )(xt, mask_row)

    return jnp.transpose(out_t, (0, 2, 1)), keep_mask


# speedup vs baseline: 1.0135x; 1.0135x over previous
import functools

import jax
import jax.numpy as jnp
from jax import lax
from jax.experimental import pallas as pl
from jax.experimental.pallas import tpu as pltpu
from jax.experimental.pallas import tpu_sc as plsc

_DROP_PROB = 0.1
_MIN_KEEP = 1


def _select_kernel(imp_ref, noise_ref, keep_ref, scaled_ref, *, k, scale):
    rows, lanes = imp_ref.shape
    n = rows * lanes
    score = 1.0 / (imp_ref[...] + 1e-8) + noise_ref[...]
    bits = jax.lax.bitcast_convert_type(score, jnp.int32)

    def _bits_body(_, carry):
        lo, hi = carry
        mid = lo + (hi - lo) // 2
        cnt = jnp.sum((bits <= mid).astype(jnp.int32))
        ge = cnt >= k
        return jnp.where(ge, lo, mid + 1), jnp.where(ge, mid, hi)

    t, _ = jax.lax.fori_loop(
        0, 31, _bits_body, (jnp.int32(0), jnp.int32(2**31 - 1))
    )

    n_less = jnp.sum((bits < t).astype(jnp.int32))
    rem = k - n_less
    eq = bits == t
    idx = (
        jax.lax.broadcasted_iota(jnp.int32, (rows, lanes), 0) * lanes
        + jax.lax.broadcasted_iota(jnp.int32, (rows, lanes), 1)
    )

    def _idx_body(_, carry):
        lo, hi = carry
        mid = lo + (hi - lo) // 2
        cnt = jnp.sum((eq & (idx < mid)).astype(jnp.int32))
        ge = cnt >= rem
        return jnp.where(ge, lo, mid + 1), jnp.where(ge, mid, hi)

    m, _ = jax.lax.fori_loop(0, 16, _idx_body, (jnp.int32(0), jnp.int32(n)))

    keep = (bits < t) | (eq & (idx < m))
    keep_f = keep.astype(keep_ref.dtype)
    keep_ref[...] = keep_f
    scaled_ref[...] = keep_f * scale


def _sc_select_body(imp_hbm, noise_hbm, keep_hbm, scaled_hbm,
                    imp_v, noise_v, bits_v, hist_v, hist256_v, ghist_v,
                    keep_v, scaled_v, row_v, shist, sem_a, sem_b,
                    *, n, k, scale, tiles):
    sl = n // tiles
    nv = sl // 16
    cid = lax.axis_index("c")
    wid = lax.axis_index("s")
    base_off = wid * sl
    lane = lax.iota(jnp.int32, 16)
    ones_i = jnp.ones((16,), jnp.int32)

    cp_a = pltpu.async_copy(imp_hbm.at[pl.ds(base_off, sl)], imp_v, sem_a)
    cp_b = pltpu.async_copy(noise_hbm.at[pl.ds(base_off, sl)], noise_v, sem_b)
    cp_a.wait()
    cp_b.wait()

    def _compute_bits(j, _):
        v = imp_v[pl.ds(j * 16, 16)]
        s = 1.0 / (v + 1e-8) + noise_v[pl.ds(j * 16, 16)]
        bits_v[pl.ds(j * 16, 16)] = lax.bitcast_convert_type(s, jnp.int32)
        return 0

    lax.fori_loop(0, nv, _compute_bits, 0)

    prefix = jnp.int32(0)
    base = jnp.int32(0)
    for r in range(4):
        sh = 24 - 8 * r

        def _zero_hist(j, _):
            hist_v[pl.ds(j * 16, 16)] = jnp.zeros((16,), jnp.int32)
            return 0

        lax.fori_loop(0, (16 * 256) // 16, _zero_hist, 0)

        def _hist(j, _, _sh=sh, _r=r, _prefix=prefix):
            v = bits_v[pl.ds(j * 16, 16)]
            if _r == 0:
                active = v == v
            else:
                active = lax.shift_right_logical(v, _sh + 8) == _prefix
            bin_ = jnp.bitwise_and(lax.shift_right_logical(v, _sh), 255)
            plsc.addupdate_scatter(hist_v, [lane * 256 + bin_], ones_i,
                                   mask=active)
            return 0

        lax.fori_loop(0, nv, _hist, 0)

        def _reduce_lanes(cc, _):
            acc = jnp.zeros((16,), jnp.int32)
            for l in range(16):
                acc = acc + hist_v[pl.ds(l * 256 + cc * 16, 16)]
            hist256_v[pl.ds(cc * 16, 16)] = acc
            return 0

        lax.fori_loop(0, 16, _reduce_lanes, 0)

        pltpu.sync_copy(hist256_v, shist.at[wid])
        plsc.subcore_barrier()
        pltpu.sync_copy(shist, row_v)

        def _reduce_tiles(cc, _):
            acc = jnp.zeros((16,), jnp.int32)
            for t in range(tiles):
                acc = acc + row_v[t, pl.ds(cc * 16, 16)]
            ghist_v[pl.ds(cc * 16, 16)] = acc
            return 0

        lax.fori_loop(0, 16, _reduce_tiles, 0)

        def _scan(cc, carry):
            cum, found, bstar, below = carry
            chunk = ghist_v[pl.ds(cc * 16, 16)]
            csum = plsc.cumsum(chunk)
            tot = jnp.sum(chunk)
            meets = (base + cum + csum) >= k
            p = jnp.sum(jnp.where(meets, 0, 1))
            before = jnp.sum(jnp.where(meets, 0, chunk))
            has = (base + cum + tot) >= k
            is_new = jnp.logical_and(has, jnp.logical_not(found))
            bstar = jnp.where(is_new, cc * 16 + p, bstar)
            below = jnp.where(is_new, cum + before, below)
            return cum + tot, jnp.logical_or(found, has), bstar, below

        _, _, bstar, below = lax.fori_loop(
            0, 16, _scan,
            (jnp.int32(0), jnp.bool_(False), jnp.int32(0), jnp.int32(0)),
        )
        prefix = prefix * 256 + bstar
        base = base + below

    t_bits = prefix
    rem = k - base

    chunk_start = (bstar // 16) * 16
    lane_sel = bstar - chunk_start
    my_chunk = hist256_v[pl.ds(chunk_start, 16)]
    local_eq = jnp.sum(jnp.where(lane == lane_sel, my_chunk, 0))
    eq_before = jnp.int32(0)
    for t in range(tiles):
        chunk_t = row_v[t, pl.ds(chunk_start, 16)]
        cnt_t = jnp.sum(jnp.where(lane == lane_sel, chunk_t, 0))
        eq_before = eq_before + jnp.where(jnp.int32(t) < wid, cnt_t, 0)

    quota = jnp.maximum(jnp.minimum(rem - eq_before, local_eq), 0)

    def _emit(j, carry):
        v = bits_v[pl.ds(j * 16, 16)]
        eq = v == t_bits
        less = v < t_bits
        eqi = jnp.where(eq, 1, 0)
        csum = plsc.cumsum(eqi)
        keep_extra = jnp.logical_and(eq, (carry + csum) <= quota)
        keep_f = jnp.where(jnp.logical_or(less, keep_extra),
                           jnp.float32(1.0), jnp.float32(0.0))
        keep_v[pl.ds(j * 16, 16)] = keep_f
        scaled_v[pl.ds(j * 16, 16)] = keep_f * scale
        return carry + jnp.sum(eqi)

    lax.fori_loop(0, nv, _emit, jnp.int32(0))

    @pl.when(cid == 0)
    def _():
        pltpu.sync_copy(keep_v, keep_hbm.at[pl.ds(base_off, sl)])
        pltpu.sync_copy(scaled_v, scaled_hbm.at[pl.ds(base_off, sl)])


def _sc_select(importance, noise, *, n, k, scale):
    tiles = 16
    sl = n // tiles
    mesh = plsc.VectorSubcoreMesh(core_axis_name="c", subcore_axis_name="s")
    f32, i32 = jnp.float32, jnp.int32
    body = functools.partial(_sc_select_body, n=n, k=k, scale=scale,
                             tiles=tiles)
    return pl.kernel(
        body,
        mesh=mesh,
        compiler_params=pltpu.CompilerParams(needs_layout_passes=False),
        out_type=(
            jax.ShapeDtypeStruct((n,), f32),
            jax.ShapeDtypeStruct((n,), f32),
        ),
        scratch_types=[
            pltpu.VMEM((sl,), f32),
            pltpu.VMEM((sl,), f32),
            pltpu.VMEM((sl,), i32),
            pltpu.VMEM((16 * 256,), i32),
            pltpu.VMEM((256,), i32),
            pltpu.VMEM((256,), i32),
            pltpu.VMEM((sl,), f32),
            pltpu.VMEM((sl,), f32),
            pltpu.VMEM((tiles, 256), i32),
            pltpu.VMEM_SHARED((tiles, 256), i32),
            pltpu.SemaphoreType.DMA,
            pltpu.SemaphoreType.DMA,
        ],
    )(importance, noise)


def _apply_kernel(x_ref, m_ref, o_ref):
    o_ref[...] = x_ref[...] * m_ref[0:1, :][:, None, :]


def kernel(x, importance):
    b, n, c = x.shape
    num_keep = max(_MIN_KEEP, int(n * (1.0 - _DROP_PROB)))
    scale = n / num_keep
    noise = (
        jax.random.uniform(jax.random.key(42), importance.shape,
                           dtype=importance.dtype)
        * 0.5
    )
    keep_mask, scaled = _sc_select(importance, noise, n=n, k=num_keep,
                                   scale=scale)
    xt = jnp.transpose(x, (0, 2, 1))
    mask_row = jnp.broadcast_to(scaled[None, :], (8, n))

    w = n
    out_t = pl.pallas_call(
        _apply_kernel,
        grid=(n // w, b // 4),
        in_specs=[
            pl.BlockSpec((4, c, w), lambda j, i: (i, 0, j)),
            pl.BlockSpec((8, w), lambda j, i: (0, j)),
        ],
        out_specs=pl.BlockSpec((4, c, w), lambda j, i: (i, 0, j)),
        out_shape=jax.ShapeDtypeStruct((b, c, n), x.dtype),
        compiler_params=pltpu.CompilerParams(
            dimension_semantics=("arbitrary", "arbitrary"),
        ),
    )(xt, mask_row)

    return jnp.transpose(out_t, (0, 2, 1)), keep_mask
